# baseline (speedup 1.0000x reference)
import functools

import jax
import jax.numpy as jnp
from jax import lax
from jax.experimental import pallas as pl
from jax.experimental.pallas import tpu as pltpu
from jax.experimental.pallas import tpu_sc as plsc

_EMB_DIM = 64
_NOISE_DIM = 32
_NUM_CORES = 2
_NUM_SUBCORES = 16
_NW = _NUM_CORES * _NUM_SUBCORES
_CHUNK = 128


def _make_gather(batch: int, emb_dim: int):
  b_per_w = batch // _NW
  n_chunks = b_per_w // _CHUNK
  mesh = plsc.VectorSubcoreMesh(
      core_axis_name="c", subcore_axis_name="s",
      num_cores=_NUM_CORES, num_subcores=_NUM_SUBCORES)

  @functools.partial(
      pl.kernel,
      out_type=jax.ShapeDtypeStruct((batch, emb_dim), jnp.float32),
      mesh=mesh,
      scratch_types=[
          pltpu.VMEM((n_chunks, _CHUNK), jnp.int32),
          pltpu.VMEM((b_per_w, emb_dim), jnp.float32),
          pltpu.SemaphoreType.DMA,
      ],
      compiler_params=pltpu.CompilerParams(use_tc_tiling_on_sc=False),
  )
  def gather_kernel(labels_hbm, table_hbm, out_hbm, idx_v, rows_v, sem):
    wid = lax.axis_index("s") * _NUM_CORES + lax.axis_index("c")
    base = wid * n_chunks
    pltpu.sync_copy(labels_hbm.at[pl.ds(base, n_chunks)], idx_v)
    for j in range(n_chunks):
      pltpu.async_copy(
          table_hbm.at[idx_v.at[j]],
          rows_v.at[pl.ds(j * _CHUNK, _CHUNK)],
          sem,
      )
    for j in range(n_chunks):
      pltpu.make_async_copy(
          table_hbm.at[idx_v.at[j]],
          rows_v.at[pl.ds(j * _CHUNK, _CHUNK)],
          sem,
      ).wait()
    pltpu.sync_copy(rows_v, out_hbm.at[pl.ds(wid * b_per_w, b_per_w)])

  return gather_kernel


def _post_kernel(embs_ref, noise_ref, out_ref, *, emb_dim, noise_dim):
  e = embs_ref[...]
  mean = jnp.mean(e, axis=-1, keepdims=True)
  var = jnp.sum((e - mean) ** 2, axis=-1, keepdims=True) / (emb_dim - 1)
  std = jnp.sqrt(var)
  z = std * noise_ref[...] + mean
  out_ref[...] = jnp.concatenate((z, e), axis=-1)


def kernel(labels, table):
  batch = labels.shape[0]
  emb_dim = table.shape[1]
  labels = labels.astype(jnp.int32)
  noise = jax.random.normal(jax.random.key(42), (batch, _NOISE_DIM),
                            dtype=jnp.float32)

  labels2d = labels.reshape(batch // _CHUNK, _CHUNK)
  embs = _make_gather(batch, emb_dim)(labels2d, table)

  blk = 2048
  out = pl.pallas_call(
      functools.partial(_post_kernel, emb_dim=emb_dim, noise_dim=_NOISE_DIM),
      grid=(batch // blk,),
      in_specs=[
          pl.BlockSpec((blk, emb_dim), lambda i: (i, 0)),
          pl.BlockSpec((blk, _NOISE_DIM), lambda i: (i, 0)),
      ],
      out_specs=pl.BlockSpec((blk, emb_dim + _NOISE_DIM), lambda i: (i, 0)),
      out_shape=jax.ShapeDtypeStruct((batch, emb_dim + _NOISE_DIM),
                                     jnp.float32),
  )(embs, noise)
  return out


# speedup vs baseline: 2.2467x; 2.2467x over previous
import functools

import jax
import jax.numpy as jnp
from jax import lax
from jax.experimental import pallas as pl
from jax.experimental.pallas import tpu as pltpu
from jax.experimental.pallas import tpu_sc as plsc

_EMB_DIM = 64
_NOISE_DIM = 32
_NUM_CORES = 2
_NUM_SUBCORES = 16
_NW = _NUM_CORES * _NUM_SUBCORES
_CHUNK = 32
_NBUF = 2


_NSLOT = 8


def _make_gather(batch: int, emb_dim: int):
  b_per_w = batch // _NW
  n_groups = b_per_w // _NSLOT
  mesh = plsc.VectorSubcoreMesh(
      core_axis_name="c", subcore_axis_name="s",
      num_cores=_NUM_CORES, num_subcores=_NUM_SUBCORES)

  @functools.partial(
      pl.kernel,
      out_type=jax.ShapeDtypeStruct((batch, emb_dim), jnp.float32),
      mesh=mesh,
      scratch_types=[
          pltpu.VMEM((b_per_w + 16,), jnp.int32),
          pltpu.VMEM((_NSLOT, 8, emb_dim), jnp.float32),
          pltpu.VMEM((b_per_w, emb_dim), jnp.float32),
          pltpu.SemaphoreType.DMA((_NSLOT,)),
      ],
  )
  def gather_kernel(labels_hbm, table_hbm, out_hbm, lab_v,
                    buf_v, rows_v, sems):
    wid = lax.axis_index("s") * _NUM_CORES + lax.axis_index("c")
    base = wid * b_per_w
    pltpu.sync_copy(labels_hbm.at[pl.ds(base, b_per_w)],
                    lab_v.at[pl.ds(0, b_per_w)])

    def _label(j):
      return lab_v[pl.ds(j, 16)][0]

    def _issue(j, s):
      g = lax.shift_right_logical(_label(j), 3)
      pltpu.async_copy(table_hbm.at[g], buf_v.at[s], sems.at[s])

    def _wait(s):
      pltpu.make_async_copy(table_hbm.at[0], buf_v.at[s], sems.at[s]).wait()

    def _extract(j, s):
      r = _label(j) & 7
      for k in range(emb_dim // 16):
        sl = pl.ds(k * 16, 16)
        rows_v[j, sl] = buf_v[s, r, sl]

    for s in range(_NSLOT):
      _issue(s, s)

    @pl.loop(0, n_groups - 1)
    def _body(q):
      j0 = q * _NSLOT
      for s in range(_NSLOT):
        _wait(s)
        _extract(j0 + s, s)
        _issue(j0 + s + _NSLOT, s)

    for s in range(_NSLOT):
      _wait(s)
      _extract((n_groups - 1) * _NSLOT + s, s)

    pltpu.sync_copy(rows_v, out_hbm.at[pl.ds(base, b_per_w)])

  return gather_kernel


def _post_kernel(embs_ref, noise_ref, out_ref, *, emb_dim, noise_dim):
  del noise_dim
  e = embs_ref[...]
  mean = jnp.mean(e, axis=-1, keepdims=True)
  var = jnp.sum((e - mean) ** 2, axis=-1, keepdims=True) / (emb_dim - 1)
  std = jnp.sqrt(var)
  z = std * noise_ref[...] + mean
  out_ref[...] = jnp.concatenate((z, e), axis=-1)


def kernel(labels, table):
  batch = labels.shape[0]
  n_rows, emb_dim = table.shape
  labels = labels.astype(jnp.int32)
  noise = jax.random.normal(jax.random.key(42), (batch, _NOISE_DIM),
                            dtype=jnp.float32)

  table3 = table.reshape(n_rows // 8, 8, emb_dim)
  embs = _make_gather(batch, emb_dim)(labels, table3)

  blk = 2048
  out = pl.pallas_call(
      functools.partial(_post_kernel, emb_dim=emb_dim, noise_dim=_NOISE_DIM),
      grid=(batch // blk,),
      in_specs=[
          pl.BlockSpec((blk, emb_dim), lambda i: (i, 0)),
          pl.BlockSpec((blk, _NOISE_DIM), lambda i: (i, 0)),
      ],
      out_specs=pl.BlockSpec((blk, emb_dim + _NOISE_DIM), lambda i: (i, 0)),
      out_shape=jax.ShapeDtypeStruct((batch, emb_dim + _NOISE_DIM),
                                     jnp.float32),
  )(embs, noise)
  return out
